# Initial kernel scaffold; baseline (speedup 1.0000x reference)
import jax
import jax.numpy as jnp
from jax.experimental import pallas as pl


def kernel(x_in_play, x_out_of_play, x_destination, ei_ip_nb_ip, ei_ip_nb_de, ei_de_nb_ip, ei_de_nb_de, ei_ip_rnb_ip, ei_de_rnb_ip, ei_ip_rnb_de, ei_de_rnb_de, ei_ip_mv_de, ei_op_mv_de, ei_de_rmv_ip, ei_de_rmv_op, ea_ip_mv_de, ea_op_mv_de, ea_de_rmv_ip, ea_de_rmv_op, move_to_action_indices, params):
    raise NotImplementedError("write your pallas kernel here")


# speedup vs baseline: 1.0568x; 1.0568x over previous
import functools

import jax
import jax.numpy as jnp
from jax.experimental import pallas as pl

HID = 128
HEADS = 4
HD = HID // HEADS
NF = 10
NN = {'ip': 10000, 'op': 2000, 'de': 10000}
NB = [('ip', 'ei_ip_nb_ip', 'ip'), ('ip', 'ei_ip_nb_de', 'de'), ('de', 'ei_de_nb_ip', 'ip'), ('de', 'ei_de_nb_de', 'de'), ('ip', 'ei_ip_rnb_ip', 'ip'), ('de', 'ei_de_rnb_ip', 'ip'), ('ip', 'ei_ip_rnb_de', 'de'), ('de', 'ei_de_rnb_de', 'de')]
MV = [('ip', 'ei_ip_mv_de', 'de', 'ea_ip_mv_de'), ('op', 'ei_op_mv_de', 'de', 'ea_op_mv_de'), ('de', 'ei_de_rmv_ip', 'ip', 'ea_de_rmv_ip'), ('de', 'ei_de_rmv_op', 'op', 'ea_de_rmv_op')]
EPS = 1e-5
NUM_ACTIONS = 120000


def _mm_body(x_ref, w_ref, o_ref):
    o_ref[...] = jnp.dot(x_ref[...], w_ref[...],
                         preferred_element_type=jnp.float32)


@functools.partial(jax.jit, static_argnames=("bm",))
def _matmul(x, w, bm=512):
    m, k = x.shape
    k2, n = w.shape
    pad = (-m) % bm
    if pad:
        x = jnp.pad(x, ((0, pad), (0, 0)))
    mp = x.shape[0]
    out = pl.pallas_call(
        _mm_body,
        grid=(mp // bm,),
        in_specs=[pl.BlockSpec((bm, k), lambda i: (i, 0)),
                  pl.BlockSpec((k, n), lambda i: (0, 0))],
        out_specs=pl.BlockSpec((bm, n), lambda i: (i, 0)),
        out_shape=jax.ShapeDtypeStruct((mp, n), jnp.float32),
    )(x, w)
    return out[:m]


def _gat(xs, xd, ei, p, n_dst, ee=None):
    hs = xs @ p['Ws']
    hd = xd @ p['Wd']
    src, dst = ei[0], ei[1]
    m = hs[src] + hd[dst]
    if ee is not None:
        m = m + ee @ p['We']
    m = m.reshape(-1, HEADS, HD)
    logit = jnp.einsum('ehc,hc->eh', jax.nn.leaky_relu(m, 0.2), p['att'])
    ex = jnp.exp(logit)
    den = jax.ops.segment_sum(ex, dst, num_segments=n_dst)
    alpha = ex / (den[dst] + 1e-16)
    msg = hs[src].reshape(-1, HEADS, HD) * alpha[:, :, None]
    out = jax.ops.segment_sum(msg, dst, num_segments=n_dst).reshape(n_dst, HID)
    return out + p['b']


def kernel(x_in_play, x_out_of_play, x_destination,
           ei_ip_nb_ip, ei_ip_nb_de, ei_de_nb_ip, ei_de_nb_de,
           ei_ip_rnb_ip, ei_de_rnb_ip, ei_ip_rnb_de, ei_de_rnb_de,
           ei_ip_mv_de, ei_op_mv_de, ei_de_rmv_ip, ei_de_rmv_op,
           ea_ip_mv_de, ea_op_mv_de, ea_de_rmv_ip, ea_de_rmv_op,
           move_to_action_indices, params):
    eis = {'ei_ip_nb_ip': ei_ip_nb_ip, 'ei_ip_nb_de': ei_ip_nb_de,
           'ei_de_nb_ip': ei_de_nb_ip, 'ei_de_nb_de': ei_de_nb_de,
           'ei_ip_rnb_ip': ei_ip_rnb_ip, 'ei_de_rnb_ip': ei_de_rnb_ip,
           'ei_ip_rnb_de': ei_ip_rnb_de, 'ei_de_rnb_de': ei_de_rnb_de,
           'ei_ip_mv_de': ei_ip_mv_de, 'ei_op_mv_de': ei_op_mv_de,
           'ei_de_rmv_ip': ei_de_rmv_ip, 'ei_de_rmv_op': ei_de_rmv_op}
    eas = {'ea_ip_mv_de': ea_ip_mv_de, 'ea_op_mv_de': ea_op_mv_de,
           'ea_de_rmv_ip': ea_de_rmv_ip, 'ea_de_rmv_op': ea_de_rmv_op}
    lin = lambda x, wb: x @ wb[0] + wb[1]
    x = {'ip': jax.nn.relu(_matmul(x_in_play, params['emb']['ip'][0]) + params['emb']['ip'][1]),
         'op': jax.nn.relu(lin(x_out_of_play, params['emb']['op'])),
         'de': jax.nn.relu(lin(x_destination, params['emb']['de']))}
    ee = {aname: lin(eas[aname], params['mv_emb']) for _, _, _, aname in MV}
    for lp in params['layers']:
        agg = {t: jnp.zeros((NN[t], HID), jnp.float32) for t in ['ip', 'op', 'de']}
        for s, name, d in NB:
            agg[d] = agg[d] + _gat(x[s], x[d], eis[name], lp['conv'][name], NN[d])
        for s, name, d, aname in MV:
            agg[d] = agg[d] + _gat(x[s], x[d], eis[name], lp['conv'][name], NN[d], ee[aname])
        for t in ['ip', 'op', 'de']:
            h = jax.nn.relu(agg[t]) + x[t]
            g, b = lp['bn'][t]
            x[t] = h / jnp.sqrt(1.0 + EPS) * g + b
    feats = []
    attrs = []
    for s, name, d, aname in MV:
        ei = eis[name]
        feats.append((x[s][ei[0]] + x[d][ei[1]]) / 2.0)
        attrs.append(eas[aname])
    f = jnp.concatenate(feats, axis=0)
    a = jnp.concatenate(attrs, axis=0)
    w1, b1, w2, b2, w3, b3 = params['head']
    v = jnp.tanh(jax.nn.relu(jax.nn.relu(f @ w1 + b1) @ w2 + b2) @ w3 + b3)[:, 0]
    masked = jnp.where(a[:, 0] == 1.0, v, -jnp.inf)
    av = jnp.full((NUM_ACTIONS,), -jnp.inf, jnp.float32).at[move_to_action_indices].set(masked)
    return av[None, :], jnp.max(av).reshape(1, 1)


# speedup vs baseline: 17.0064x; 16.0928x over previous
import functools

import numpy as np
import jax
import jax.numpy as jnp
from jax import lax
from jax.experimental import pallas as pl
from jax.experimental.pallas import tpu as pltpu
from jax.experimental.pallas import tpu_sc as plsc

HID = 128
HEADS = 4
HD = HID // HEADS
NN = {'ip': 10000, 'op': 2000, 'de': 10000}
NB = [('ip', 'ei_ip_nb_ip', 'ip'), ('ip', 'ei_ip_nb_de', 'de'),
      ('de', 'ei_de_nb_ip', 'ip'), ('de', 'ei_de_nb_de', 'de'),
      ('ip', 'ei_ip_rnb_ip', 'ip'), ('de', 'ei_de_rnb_ip', 'ip'),
      ('ip', 'ei_ip_rnb_de', 'de'), ('de', 'ei_de_rnb_de', 'de')]
MV = [('ip', 'ei_ip_mv_de', 'de', 'ea_ip_mv_de'),
      ('op', 'ei_op_mv_de', 'de', 'ea_op_mv_de'),
      ('de', 'ei_de_rmv_ip', 'ip', 'ea_de_rmv_ip'),
      ('de', 'ei_de_rmv_op', 'op', 'ea_de_rmv_op')]
EPS = 1e-5
NUM_ACTIONS = 120000

ALL_ETYPES = [(s, name, d, None) for s, name, d in NB] + \
             [(s, name, d, aname) for s, name, d, aname in MV]

NC = 2
NS = 16
NW = NC * NS
CHUNK = 128
UNIT = NW * CHUNK

BM = 512
BN = 512
NDP = {'ip': 10240, 'op': 2048, 'de': 10240}

_SEL = np.zeros((16, HID), np.float32)
for _h in range(HEADS):
    _SEL[_h, _h * HD:(_h + 1) * HD] = 1.0

_EYE = np.zeros((16, HID), np.float32)
for _h in range(16):
    _EYE[_h, _h] = 1.0
_SELP = np.zeros((HID, HID), np.float32)
_SELP[:16, :] = _SEL


def _mesh():
    return plsc.VectorSubcoreMesh(core_axis_name="c", subcore_axis_name="s",
                                  num_cores=NC, num_subcores=NS)


def _pad_edges(n):
    return ((n + UNIT - 1) // UNIT) * UNIT


@functools.lru_cache(None)
def _gather2(ep):
    bw = ep // NW
    nch = bw // CHUNK

    def body(hs_hbm, hd_hbm, si_hbm, di_hbm, ghs_hbm, ghd_hbm,
             ia_v, ib_v, ra_v, rb_v, sema, semb):
        wid = lax.axis_index("s") * NC + lax.axis_index("c")
        base = wid * bw

        @pl.loop(0, nch)
        def _(i):
            off = base + i * CHUNK
            pltpu.sync_copy(si_hbm.at[pl.ds(off, CHUNK)], ia_v)
            pltpu.sync_copy(di_hbm.at[pl.ds(off, CHUNK)], ib_v)
            cpa = pltpu.async_copy(hs_hbm.at[ia_v], ra_v, sema)
            cpb = pltpu.async_copy(hd_hbm.at[ib_v], rb_v, semb)
            cpa.wait()
            cpb.wait()
            pltpu.sync_copy(ra_v, ghs_hbm.at[pl.ds(off, CHUNK)])
            pltpu.sync_copy(rb_v, ghd_hbm.at[pl.ds(off, CHUNK)])

    return pl.kernel(
        body,
        out_type=[jax.ShapeDtypeStruct((ep, HID), jnp.float32),
                  jax.ShapeDtypeStruct((ep, HID), jnp.float32)],
        mesh=_mesh(),
        scratch_types=[pltpu.VMEM((CHUNK,), jnp.int32),
                       pltpu.VMEM((CHUNK,), jnp.int32),
                       pltpu.VMEM((CHUNK, HID), jnp.float32),
                       pltpu.VMEM((CHUNK, HID), jnp.float32),
                       pltpu.SemaphoreType.DMA,
                       pltpu.SemaphoreType.DMA],
    )


@functools.lru_cache(None)
def _scatter2(ep, nd):
    bw = ep // NW
    nch = bw // CHUNK
    nz = nd // NS
    nzc = nz // CHUNK

    def body(wm_hbm, ex_hbm, di_hbm, z_hbm, pn_hbm, pd_hbm,
             idx_v, wm_v, zv, tn):
        c = lax.axis_index("c")
        s = lax.axis_index("s")
        wid = s * NC + c
        base = wid * bw
        pltpu.sync_copy(z_hbm, zv)

        for phase in range(2):
            src_hbm = wm_hbm if phase == 0 else ex_hbm
            out_hbm = pn_hbm if phase == 0 else pd_hbm

            @pl.loop(0, nzc)
            def _(j):
                pltpu.sync_copy(zv, tn.at[pl.ds(s * nz + j * CHUNK, CHUNK)])

            plsc.subcore_barrier()

            @pl.loop(0, nch)
            def _(i):
                off = base + i * CHUNK
                pltpu.sync_copy(di_hbm.at[pl.ds(off, CHUNK)], idx_v)
                pltpu.sync_copy(src_hbm.at[pl.ds(off, CHUNK)], wm_v)
                pltpu.sync_copy(wm_v, tn.at[idx_v], add=True)

            plsc.subcore_barrier()

            @pl.loop(0, nzc)
            def _(j):
                o = s * nz + j * CHUNK
                pltpu.sync_copy(tn.at[pl.ds(o, CHUNK)], wm_v)
                pltpu.sync_copy(wm_v, out_hbm.at[c, pl.ds(o, CHUNK)])

            plsc.subcore_barrier()

    return pl.kernel(
        body,
        out_type=[jax.ShapeDtypeStruct((NC, nd, HID), jnp.float32),
                  jax.ShapeDtypeStruct((NC, nd, HID), jnp.float32)],
        mesh=_mesh(),
        scratch_types=[pltpu.VMEM((CHUNK,), jnp.int32),
                       pltpu.VMEM((CHUNK, HID), jnp.float32),
                       pltpu.VMEM((CHUNK, HID), jnp.float32),
                       pltpu.VMEM_SHARED((nd, HID), jnp.float32)],
    )


@functools.lru_cache(None)
def _linear_tc(m, k, n, act, bm):
    nblk = m // bm

    def body(x_ref, w_ref, b_ref, o_ref):
        r = jnp.dot(x_ref[...], w_ref[...], preferred_element_type=jnp.float32)
        r = r + b_ref[...]
        if act == 'relu':
            r = jnp.maximum(r, 0.0)
        o_ref[...] = r

    return pl.pallas_call(
        body,
        grid=(nblk,),
        in_specs=[pl.BlockSpec((bm, k), lambda i: (i, 0)),
                  pl.BlockSpec((k, n), lambda i: (0, 0)),
                  pl.BlockSpec((1, n), lambda i: (0, 0))],
        out_specs=pl.BlockSpec((bm, n), lambda i: (i, 0)),
        out_shape=jax.ShapeDtypeStruct((m, n), jnp.float32),
    )


def _linear(x, w, b=None, act=None, bm=BM):
    m, k = x.shape
    n = w.shape[1]
    pad = (-m) % bm
    if pad:
        x = jnp.pad(x, ((0, pad), (0, 0)))
    if b is None:
        b = jnp.zeros((n,), jnp.float32)
    out = _linear_tc(x.shape[0], k, n, act, bm)(x, w, b.reshape(1, n))
    return out[:m] if pad else out


@functools.lru_cache(None)
def _edge_tc(ep, e_real, mv):
    nblk = ep // BM

    def body(*refs):
        if mv:
            ghs, ghd, ea, a_r, sel_r, eye_r, g0_r, gd_r, wm_o, ex_o = refs
        else:
            ghs, ghd, a_r, sel_r, eye_r, wm_o, ex_o = refs
        m = ghs[...] + ghd[...]
        if mv:
            m = m + ea[...] * gd_r[...] + g0_r[...]
        lr = jnp.where(m >= 0.0, m, 0.2 * m)
        ex16 = jnp.exp(jnp.dot(lr, a_r[...], preferred_element_type=jnp.float32))
        row = pl.program_id(0) * BM + lax.broadcasted_iota(jnp.int32, (BM, 1), 0)
        ex16 = jnp.where(row < e_real, ex16, 0.0)
        ex_o[...] = jnp.dot(ex16, eye_r[...], preferred_element_type=jnp.float32)
        wm_o[...] = ghs[...] * jnp.dot(ex16, sel_r[...],
                                       preferred_element_type=jnp.float32)

    ins = [pl.BlockSpec((BM, HID), lambda i: (i, 0)),
           pl.BlockSpec((BM, HID), lambda i: (i, 0))]
    if mv:
        ins.append(pl.BlockSpec((BM, 1), lambda i: (i, 0)))
    ins += [pl.BlockSpec((HID, 16), lambda i: (0, 0)),
            pl.BlockSpec((16, HID), lambda i: (0, 0)),
            pl.BlockSpec((16, HID), lambda i: (0, 0))]
    if mv:
        ins += [pl.BlockSpec((1, HID), lambda i: (0, 0)),
                pl.BlockSpec((1, HID), lambda i: (0, 0))]
    return pl.pallas_call(
        body,
        grid=(nblk,),
        in_specs=ins,
        out_specs=[pl.BlockSpec((BM, HID), lambda i: (i, 0)),
                   pl.BlockSpec((BM, HID), lambda i: (i, 0))],
        out_shape=[jax.ShapeDtypeStruct((ep, HID), jnp.float32),
                   jax.ShapeDtypeStruct((ep, HID), jnp.float32)],
    )


@functools.lru_cache(None)
def _combine_tc(n, k):
    nblk = n // BN

    def body(pn, pd_, x, bsum, gs, bb, sel, o):
        acc = None
        for e in range(k):
            num = pn[e, 0] + pn[e, 1]
            den = pd_[e, 0] + pd_[e, 1]
            denb = jnp.dot(den, sel[...],
                           preferred_element_type=jnp.float32) + 1e-16
            t = num / denb
            acc = t if acc is None else acc + t
        h = jnp.maximum(acc + bsum[...], 0.0) + x[...]
        o[...] = h * gs[...] + bb[...]

    return pl.pallas_call(
        body,
        grid=(nblk,),
        in_specs=[pl.BlockSpec((k, NC, BN, HID), lambda i: (0, 0, i, 0)),
                  pl.BlockSpec((k, NC, BN, HID), lambda i: (0, 0, i, 0)),
                  pl.BlockSpec((BN, HID), lambda i: (i, 0)),
                  pl.BlockSpec((1, HID), lambda i: (0, 0)),
                  pl.BlockSpec((1, HID), lambda i: (0, 0)),
                  pl.BlockSpec((1, HID), lambda i: (0, 0)),
                  pl.BlockSpec((HID, HID), lambda i: (0, 0))],
        out_specs=pl.BlockSpec((BN, HID), lambda i: (i, 0)),
        out_shape=jax.ShapeDtypeStruct((n, HID), jnp.float32),
    )


@functools.lru_cache(None)
def _head_tc(ep):
    nblk = ep // BM

    def body(g0, g1, w1, b1, w2, b2, w3, b3, o):
        f = (g0[...] + g1[...]) * 0.5
        h1 = jnp.maximum(jnp.dot(f, w1[...],
                                 preferred_element_type=jnp.float32) + b1[...], 0.0)
        h2 = jnp.maximum(jnp.dot(h1, w2[...],
                                 preferred_element_type=jnp.float32) + b2[...], 0.0)
        o[...] = jnp.tanh(jnp.dot(h2, w3[...],
                                  preferred_element_type=jnp.float32) + b3[...])

    return pl.pallas_call(
        body,
        grid=(nblk,),
        in_specs=[pl.BlockSpec((BM, HID), lambda i: (i, 0)),
                  pl.BlockSpec((BM, HID), lambda i: (i, 0)),
                  pl.BlockSpec((HID, 64), lambda i: (0, 0)),
                  pl.BlockSpec((1, 64), lambda i: (0, 0)),
                  pl.BlockSpec((64, 32), lambda i: (0, 0)),
                  pl.BlockSpec((1, 32), lambda i: (0, 0)),
                  pl.BlockSpec((32, 8), lambda i: (0, 0)),
                  pl.BlockSpec((1, 8), lambda i: (0, 0))],
        out_specs=pl.BlockSpec((BM, 8), lambda i: (i, 0)),
        out_shape=jax.ShapeDtypeStruct((ep, 8), jnp.float32),
    )


def _att_mat(att):
    cols = []
    for h in range(HEADS):
        cols.append(jnp.zeros((HD, 16), jnp.float32).at[:, h].set(att[h]))
    return jnp.concatenate(cols, axis=0)


def kernel(x_in_play, x_out_of_play, x_destination,
           ei_ip_nb_ip, ei_ip_nb_de, ei_de_nb_ip, ei_de_nb_de,
           ei_ip_rnb_ip, ei_de_rnb_ip, ei_ip_rnb_de, ei_de_rnb_de,
           ei_ip_mv_de, ei_op_mv_de, ei_de_rmv_ip, ei_de_rmv_op,
           ea_ip_mv_de, ea_op_mv_de, ea_de_rmv_ip, ea_de_rmv_op,
           move_to_action_indices, params):
    eis = {'ei_ip_nb_ip': ei_ip_nb_ip, 'ei_ip_nb_de': ei_ip_nb_de,
           'ei_de_nb_ip': ei_de_nb_ip, 'ei_de_nb_de': ei_de_nb_de,
           'ei_ip_rnb_ip': ei_ip_rnb_ip, 'ei_de_rnb_ip': ei_de_rnb_ip,
           'ei_ip_rnb_de': ei_ip_rnb_de, 'ei_de_rnb_de': ei_de_rnb_de,
           'ei_ip_mv_de': ei_ip_mv_de, 'ei_op_mv_de': ei_op_mv_de,
           'ei_de_rmv_ip': ei_de_rmv_ip, 'ei_de_rmv_op': ei_de_rmv_op}
    eas = {'ea_ip_mv_de': ea_ip_mv_de, 'ea_op_mv_de': ea_op_mv_de,
           'ea_de_rmv_ip': ea_de_rmv_ip, 'ea_de_rmv_op': ea_de_rmv_op}
    sel = jnp.asarray(_SEL)
    eye = jnp.asarray(_EYE)
    selp = jnp.asarray(_SELP)

    einfo = {}
    for s, name, d, aname in ALL_ETYPES:
        e = eis[name].shape[1]
        ep = _pad_edges(e)
        src = jnp.pad(eis[name][0].astype(jnp.int32), (0, ep - e))
        dst = jnp.pad(eis[name][1].astype(jnp.int32), (0, ep - e))
        eav = None
        if aname is not None:
            eav = jnp.pad(eas[aname].astype(jnp.float32), ((0, ep - e), (0, 0)))
        einfo[name] = (s, d, aname, e, ep, src, dst, eav)

    x = {'ip': _linear(x_in_play, *params['emb']['ip'], act='relu'),
         'op': _linear(x_out_of_play, *params['emb']['op'], act='relu'),
         'de': _linear(x_destination, *params['emb']['de'], act='relu')}

    mvw, mvb = params['mv_emb']

    for lp in params['layers']:
        conv = lp['conv']
        plan = {t: [] for t in NN}
        for s, name, d, aname in ALL_ETYPES:
            plan[s].append(('s', name, conv[name]['Ws']))
            plan[d].append(('d', name, conv[name]['Wd']))
        hmats = {}
        for t, items in plan.items():
            wcat = jnp.concatenate([w for _, _, w in items], axis=1)
            hcat = _linear(x[t], wcat)
            for j, (kind, name, _) in enumerate(items):
                hmats[(kind, name)] = hcat[:, j * HID:(j + 1) * HID]

        pn_by_dst = {t: [] for t in NN}
        pd_by_dst = {t: [] for t in NN}
        bsum = {t: jnp.zeros((HID,), jnp.float32) for t in NN}
        for s, name, d, aname in ALL_ETYPES:
            _, _, _, e, ep, srcp, dstp, eav = einfo[name]
            p = conv[name]
            ghs, ghd = _gather2(ep)(hmats[('s', name)], hmats[('d', name)],
                                    srcp, dstp)
            amat = _att_mat(p['att'])
            if aname is None:
                wm, ex = _edge_tc(ep, e, False)(ghs, ghd, amat, sel, eye)
            else:
                gd = (mvw[0] @ p['We']).reshape(1, HID)
                g0 = (mvb @ p['We']).reshape(1, HID)
                wm, ex = _edge_tc(ep, e, True)(ghs, ghd, eav, amat, sel, eye,
                                               g0, gd)
            nd = NDP[d]
            pn, pd_ = _scatter2(ep, nd)(wm, ex, dstp,
                                        jnp.zeros((CHUNK, HID), jnp.float32))
            pn_by_dst[d].append(pn)
            pd_by_dst[d].append(pd_)
            bsum[d] = bsum[d] + p['b']

        newx = {}
        for t in NN:
            k = len(pn_by_dst[t])
            g, b = lp['bn'][t]
            gs = (g / jnp.sqrt(1.0 + EPS)).reshape(1, HID)
            xp = jnp.pad(x[t], ((0, NDP[t] - NN[t]), (0, 0)))
            newx[t] = _combine_tc(NDP[t], k)(
                jnp.stack(pn_by_dst[t]), jnp.stack(pd_by_dst[t]), xp,
                bsum[t].reshape(1, HID), gs, b.reshape(1, HID), selp)[:NN[t]]
        x = newx

    w1, b1, w2, b2, w3, b3 = params['head']
    w3p = jnp.zeros((32, 8), jnp.float32).at[:, :1].set(w3)
    b3p = jnp.zeros((1, 8), jnp.float32).at[:, :1].set(b3.reshape(1, 1))
    vs = []
    attrs = []
    for s, name, d, aname in MV:
        _, _, _, e, ep, srcp, dstp, _ = einfo[name]
        g0, g1 = _gather2(ep)(x[s], x[d], srcp, dstp)
        v8 = _head_tc(ep)(g0, g1, w1, b1.reshape(1, 64), w2, b2.reshape(1, 32),
                          w3p, b3p)
        vs.append(v8[:e, 0])
        attrs.append(eas[aname])
    v = jnp.concatenate(vs, axis=0)
    a = jnp.concatenate(attrs, axis=0)
    masked = jnp.where(a[:, 0] == 1.0, v, -jnp.inf)
    av = jnp.full((NUM_ACTIONS,), -jnp.inf,
                  jnp.float32).at[move_to_action_indices].set(masked)
    return av[None, :], jnp.max(av).reshape(1, 1)


# speedup vs baseline: 17.4464x; 1.0259x over previous
import functools

import numpy as np
import jax
import jax.numpy as jnp
from jax import lax
from jax.experimental import pallas as pl
from jax.experimental.pallas import tpu as pltpu
from jax.experimental.pallas import tpu_sc as plsc

HID = 128
HEADS = 4
HD = HID // HEADS
NN = {'ip': 10000, 'op': 2000, 'de': 10000}
NB = [('ip', 'ei_ip_nb_ip', 'ip'), ('ip', 'ei_ip_nb_de', 'de'),
      ('de', 'ei_de_nb_ip', 'ip'), ('de', 'ei_de_nb_de', 'de'),
      ('ip', 'ei_ip_rnb_ip', 'ip'), ('de', 'ei_de_rnb_ip', 'ip'),
      ('ip', 'ei_ip_rnb_de', 'de'), ('de', 'ei_de_rnb_de', 'de')]
MV = [('ip', 'ei_ip_mv_de', 'de', 'ea_ip_mv_de'),
      ('op', 'ei_op_mv_de', 'de', 'ea_op_mv_de'),
      ('de', 'ei_de_rmv_ip', 'ip', 'ea_de_rmv_ip'),
      ('de', 'ei_de_rmv_op', 'op', 'ea_de_rmv_op')]
EPS = 1e-5
NUM_ACTIONS = 120000

ALL_ETYPES = [(s, name, d, None) for s, name, d in NB] + \
             [(s, name, d, aname) for s, name, d, aname in MV]

NC = 2
NS = 16
NW = NC * NS
CHUNK = 128
UNIT = NW * CHUNK

BM = 512
BN = 512
NDP = {'ip': 10240, 'op': 2048, 'de': 10240}

_SEL = np.zeros((16, HID), np.float32)
for _h in range(HEADS):
    _SEL[_h, _h * HD:(_h + 1) * HD] = 1.0

_EYE = np.zeros((16, HID), np.float32)
for _h in range(16):
    _EYE[_h, _h] = 1.0
_SELP = np.zeros((HID, HID), np.float32)
_SELP[:16, :] = _SEL


def _mesh():
    return plsc.VectorSubcoreMesh(core_axis_name="c", subcore_axis_name="s",
                                  num_cores=NC, num_subcores=NS)


def _pad_edges(n):
    return ((n + UNIT - 1) // UNIT) * UNIT


@functools.lru_cache(None)
def _gather2(ep):
    bw = ep // NW
    nch = bw // CHUNK

    def body(hs_hbm, hd_hbm, si_hbm, di_hbm, ghs_hbm, ghd_hbm,
             ia0, ia1, ib0, ib1, ra0, ra1, rb0, rb1,
             sga0, sga1, sgb0, sgb1, swa0, swa1, swb0, swb1):
        wid = lax.axis_index("s") * NC + lax.axis_index("c")
        base = wid * bw
        ia = (ia0, ia1)
        ib = (ib0, ib1)
        ra = (ra0, ra1)
        rb = (rb0, rb1)
        sga = (sga0, sga1)
        sgb = (sgb0, sgb1)
        swa = (swa0, swa1)
        swb = (swb0, swb1)
        pend_g = [None, None]
        pend_w = [None, None]
        offs = [None, None]
        for i in range(nch):
            b = i & 1
            if pend_w[b] is not None:
                pend_w[b][0].wait()
                pend_w[b][1].wait()
                pend_w[b] = None
            off = base + i * CHUNK
            pltpu.sync_copy(si_hbm.at[pl.ds(off, CHUNK)], ia[b])
            pltpu.sync_copy(di_hbm.at[pl.ds(off, CHUNK)], ib[b])
            pend_g[b] = (pltpu.async_copy(hs_hbm.at[ia[b]], ra[b], sga[b]),
                         pltpu.async_copy(hd_hbm.at[ib[b]], rb[b], sgb[b]))
            offs[b] = off
            pb = (i - 1) & 1
            if i >= 1 and pend_g[pb] is not None:
                pend_g[pb][0].wait()
                pend_g[pb][1].wait()
                pend_g[pb] = None
                po = offs[pb]
                pend_w[pb] = (
                    pltpu.async_copy(ra[pb], ghs_hbm.at[pl.ds(po, CHUNK)], swa[pb]),
                    pltpu.async_copy(rb[pb], ghd_hbm.at[pl.ds(po, CHUNK)], swb[pb]))
        lb = (nch - 1) & 1
        pend_g[lb][0].wait()
        pend_g[lb][1].wait()
        pltpu.sync_copy(ra[lb], ghs_hbm.at[pl.ds(offs[lb], CHUNK)])
        pltpu.sync_copy(rb[lb], ghd_hbm.at[pl.ds(offs[lb], CHUNK)])
        for b in range(2):
            if pend_w[b] is not None:
                pend_w[b][0].wait()
                pend_w[b][1].wait()

    return pl.kernel(
        body,
        out_type=[jax.ShapeDtypeStruct((ep, HID), jnp.float32),
                  jax.ShapeDtypeStruct((ep, HID), jnp.float32)],
        mesh=_mesh(),
        scratch_types=[pltpu.VMEM((CHUNK,), jnp.int32),
                       pltpu.VMEM((CHUNK,), jnp.int32),
                       pltpu.VMEM((CHUNK,), jnp.int32),
                       pltpu.VMEM((CHUNK,), jnp.int32),
                       pltpu.VMEM((CHUNK, HID), jnp.float32),
                       pltpu.VMEM((CHUNK, HID), jnp.float32),
                       pltpu.VMEM((CHUNK, HID), jnp.float32),
                       pltpu.VMEM((CHUNK, HID), jnp.float32)] +
                      [pltpu.SemaphoreType.DMA] * 8,
    )


@functools.lru_cache(None)
def _scatter2(ep, nd):
    bw = ep // NW
    nch = bw // CHUNK
    nz = nd // NS
    nzc = nz // CHUNK

    def body(wm_hbm, ex_hbm, di_hbm, z_hbm, pn_hbm, pd_hbm,
             ix0, ix1, wv0, wv1, zv, tn, sa0, sa1):
        c = lax.axis_index("c")
        s = lax.axis_index("s")
        wid = s * NC + c
        base = wid * bw
        ix = (ix0, ix1)
        wv = (wv0, wv1)
        sa = (sa0, sa1)
        pltpu.sync_copy(z_hbm, zv)

        for phase in range(2):
            src_hbm = wm_hbm if phase == 0 else ex_hbm
            out_hbm = pn_hbm if phase == 0 else pd_hbm

            @pl.loop(0, nzc)
            def _(j):
                pltpu.sync_copy(zv, tn.at[pl.ds(s * nz + j * CHUNK, CHUNK)])

            plsc.subcore_barrier()

            @pl.loop(0, nch)
            def _(i):
                off = base + i * CHUNK
                pltpu.sync_copy(di_hbm.at[pl.ds(off, CHUNK)], ix0)
                pltpu.sync_copy(src_hbm.at[pl.ds(off, CHUNK)], wv0)
                pltpu.sync_copy(wv0, tn.at[ix0], add=True)

            plsc.subcore_barrier()

            @pl.loop(0, nzc)
            def _(j):
                o = s * nz + j * CHUNK
                pltpu.sync_copy(tn.at[pl.ds(o, CHUNK)], wv0)
                pltpu.sync_copy(wv0, out_hbm.at[c, pl.ds(o, CHUNK)])

            plsc.subcore_barrier()

    return pl.kernel(
        body,
        out_type=[jax.ShapeDtypeStruct((NC, nd, HID), jnp.float32),
                  jax.ShapeDtypeStruct((NC, nd, HID), jnp.float32)],
        mesh=_mesh(),
        scratch_types=[pltpu.VMEM((CHUNK,), jnp.int32),
                       pltpu.VMEM((CHUNK,), jnp.int32),
                       pltpu.VMEM((CHUNK, HID), jnp.float32),
                       pltpu.VMEM((CHUNK, HID), jnp.float32),
                       pltpu.VMEM((CHUNK, HID), jnp.float32),
                       pltpu.VMEM_SHARED((nd, HID), jnp.float32),
                       pltpu.SemaphoreType.DMA,
                       pltpu.SemaphoreType.DMA],
    )


@functools.lru_cache(None)
def _linear_tc(m, k, n, act, bm):
    nblk = m // bm

    def body(x_ref, w_ref, b_ref, o_ref):
        r = jnp.dot(x_ref[...], w_ref[...], preferred_element_type=jnp.float32)
        r = r + b_ref[...]
        if act == 'relu':
            r = jnp.maximum(r, 0.0)
        o_ref[...] = r

    return pl.pallas_call(
        body,
        grid=(nblk,),
        in_specs=[pl.BlockSpec((bm, k), lambda i: (i, 0)),
                  pl.BlockSpec((k, n), lambda i: (0, 0)),
                  pl.BlockSpec((1, n), lambda i: (0, 0))],
        out_specs=pl.BlockSpec((bm, n), lambda i: (i, 0)),
        out_shape=jax.ShapeDtypeStruct((m, n), jnp.float32),
    )


def _linear(x, w, b=None, act=None, bm=BM):
    m, k = x.shape
    n = w.shape[1]
    pad = (-m) % bm
    if pad:
        x = jnp.pad(x, ((0, pad), (0, 0)))
    if b is None:
        b = jnp.zeros((n,), jnp.float32)
    out = _linear_tc(x.shape[0], k, n, act, bm)(x, w, b.reshape(1, n))
    return out[:m] if pad else out


@functools.lru_cache(None)
def _edge_tc(ep, e_real, mv):
    nblk = ep // BM

    def body(*refs):
        if mv:
            ghs, ghd, ea, a_r, sel_r, eye_r, g0_r, gd_r, wm_o, ex_o = refs
        else:
            ghs, ghd, a_r, sel_r, eye_r, wm_o, ex_o = refs
        m = ghs[...] + ghd[...]
        if mv:
            m = m + ea[...] * gd_r[...] + g0_r[...]
        lr = jnp.where(m >= 0.0, m, 0.2 * m)
        ex16 = jnp.exp(jnp.dot(lr, a_r[...], preferred_element_type=jnp.float32))
        row = pl.program_id(0) * BM + lax.broadcasted_iota(jnp.int32, (BM, 1), 0)
        ex16 = jnp.where(row < e_real, ex16, 0.0)
        ex_o[...] = jnp.dot(ex16, eye_r[...], preferred_element_type=jnp.float32)
        wm_o[...] = ghs[...] * jnp.dot(ex16, sel_r[...],
                                       preferred_element_type=jnp.float32)

    ins = [pl.BlockSpec((BM, HID), lambda i: (i, 0)),
           pl.BlockSpec((BM, HID), lambda i: (i, 0))]
    if mv:
        ins.append(pl.BlockSpec((BM, 1), lambda i: (i, 0)))
    ins += [pl.BlockSpec((HID, 16), lambda i: (0, 0)),
            pl.BlockSpec((16, HID), lambda i: (0, 0)),
            pl.BlockSpec((16, HID), lambda i: (0, 0))]
    if mv:
        ins += [pl.BlockSpec((1, HID), lambda i: (0, 0)),
                pl.BlockSpec((1, HID), lambda i: (0, 0))]
    return pl.pallas_call(
        body,
        grid=(nblk,),
        in_specs=ins,
        out_specs=[pl.BlockSpec((BM, HID), lambda i: (i, 0)),
                   pl.BlockSpec((BM, HID), lambda i: (i, 0))],
        out_shape=[jax.ShapeDtypeStruct((ep, HID), jnp.float32),
                   jax.ShapeDtypeStruct((ep, HID), jnp.float32)],
    )


@functools.lru_cache(None)
def _combine_tc(n, k):
    nblk = n // BN

    def body(pn, pd_, x, bsum, gs, bb, sel, o):
        acc = None
        for e in range(k):
            num = pn[e, 0] + pn[e, 1]
            den = pd_[e, 0] + pd_[e, 1]
            denb = jnp.dot(den, sel[...],
                           preferred_element_type=jnp.float32) + 1e-16
            t = num / denb
            acc = t if acc is None else acc + t
        h = jnp.maximum(acc + bsum[...], 0.0) + x[...]
        o[...] = h * gs[...] + bb[...]

    return pl.pallas_call(
        body,
        grid=(nblk,),
        in_specs=[pl.BlockSpec((k, NC, BN, HID), lambda i: (0, 0, i, 0)),
                  pl.BlockSpec((k, NC, BN, HID), lambda i: (0, 0, i, 0)),
                  pl.BlockSpec((BN, HID), lambda i: (i, 0)),
                  pl.BlockSpec((1, HID), lambda i: (0, 0)),
                  pl.BlockSpec((1, HID), lambda i: (0, 0)),
                  pl.BlockSpec((1, HID), lambda i: (0, 0)),
                  pl.BlockSpec((HID, HID), lambda i: (0, 0))],
        out_specs=pl.BlockSpec((BN, HID), lambda i: (i, 0)),
        out_shape=jax.ShapeDtypeStruct((n, HID), jnp.float32),
    )


@functools.lru_cache(None)
def _head_tc(ep):
    nblk = ep // BM

    def body(g0, g1, w1, b1, w2, b2, w3, b3, o):
        f = (g0[...] + g1[...]) * 0.5
        h1 = jnp.maximum(jnp.dot(f, w1[...],
                                 preferred_element_type=jnp.float32) + b1[...], 0.0)
        h2 = jnp.maximum(jnp.dot(h1, w2[...],
                                 preferred_element_type=jnp.float32) + b2[...], 0.0)
        o[...] = jnp.tanh(jnp.dot(h2, w3[...],
                                  preferred_element_type=jnp.float32) + b3[...])

    return pl.pallas_call(
        body,
        grid=(nblk,),
        in_specs=[pl.BlockSpec((BM, HID), lambda i: (i, 0)),
                  pl.BlockSpec((BM, HID), lambda i: (i, 0)),
                  pl.BlockSpec((HID, 64), lambda i: (0, 0)),
                  pl.BlockSpec((1, 64), lambda i: (0, 0)),
                  pl.BlockSpec((64, 32), lambda i: (0, 0)),
                  pl.BlockSpec((1, 32), lambda i: (0, 0)),
                  pl.BlockSpec((32, 8), lambda i: (0, 0)),
                  pl.BlockSpec((1, 8), lambda i: (0, 0))],
        out_specs=pl.BlockSpec((BM, 8), lambda i: (i, 0)),
        out_shape=jax.ShapeDtypeStruct((ep, 8), jnp.float32),
    )


def _att_mat(att):
    cols = []
    for h in range(HEADS):
        cols.append(jnp.zeros((HD, 16), jnp.float32).at[:, h].set(att[h]))
    return jnp.concatenate(cols, axis=0)


def kernel(x_in_play, x_out_of_play, x_destination,
           ei_ip_nb_ip, ei_ip_nb_de, ei_de_nb_ip, ei_de_nb_de,
           ei_ip_rnb_ip, ei_de_rnb_ip, ei_ip_rnb_de, ei_de_rnb_de,
           ei_ip_mv_de, ei_op_mv_de, ei_de_rmv_ip, ei_de_rmv_op,
           ea_ip_mv_de, ea_op_mv_de, ea_de_rmv_ip, ea_de_rmv_op,
           move_to_action_indices, params):
    eis = {'ei_ip_nb_ip': ei_ip_nb_ip, 'ei_ip_nb_de': ei_ip_nb_de,
           'ei_de_nb_ip': ei_de_nb_ip, 'ei_de_nb_de': ei_de_nb_de,
           'ei_ip_rnb_ip': ei_ip_rnb_ip, 'ei_de_rnb_ip': ei_de_rnb_ip,
           'ei_ip_rnb_de': ei_ip_rnb_de, 'ei_de_rnb_de': ei_de_rnb_de,
           'ei_ip_mv_de': ei_ip_mv_de, 'ei_op_mv_de': ei_op_mv_de,
           'ei_de_rmv_ip': ei_de_rmv_ip, 'ei_de_rmv_op': ei_de_rmv_op}
    eas = {'ea_ip_mv_de': ea_ip_mv_de, 'ea_op_mv_de': ea_op_mv_de,
           'ea_de_rmv_ip': ea_de_rmv_ip, 'ea_de_rmv_op': ea_de_rmv_op}
    sel = jnp.asarray(_SEL)
    eye = jnp.asarray(_EYE)
    selp = jnp.asarray(_SELP)

    einfo = {}
    for s, name, d, aname in ALL_ETYPES:
        e = eis[name].shape[1]
        ep = _pad_edges(e)
        src = jnp.pad(eis[name][0].astype(jnp.int32), (0, ep - e))
        dst = jnp.pad(eis[name][1].astype(jnp.int32), (0, ep - e))
        eav = None
        if aname is not None:
            eav = jnp.pad(eas[aname].astype(jnp.float32), ((0, ep - e), (0, 0)))
        einfo[name] = (s, d, aname, e, ep, src, dst, eav)

    x = {'ip': _linear(x_in_play, *params['emb']['ip'], act='relu'),
         'op': _linear(x_out_of_play, *params['emb']['op'], act='relu'),
         'de': _linear(x_destination, *params['emb']['de'], act='relu')}

    mvw, mvb = params['mv_emb']

    for lp in params['layers']:
        conv = lp['conv']
        plan = {t: [] for t in NN}
        for s, name, d, aname in ALL_ETYPES:
            plan[s].append(('s', name, conv[name]['Ws']))
            plan[d].append(('d', name, conv[name]['Wd']))
        hmats = {}
        for t, items in plan.items():
            wcat = jnp.concatenate([w for _, _, w in items], axis=1)
            hcat = _linear(x[t], wcat)
            for j, (kind, name, _) in enumerate(items):
                hmats[(kind, name)] = hcat[:, j * HID:(j + 1) * HID]

        pn_by_dst = {t: [] for t in NN}
        pd_by_dst = {t: [] for t in NN}
        bsum = {t: jnp.zeros((HID,), jnp.float32) for t in NN}
        for s, name, d, aname in ALL_ETYPES:
            _, _, _, e, ep, srcp, dstp, eav = einfo[name]
            p = conv[name]
            ghs, ghd = _gather2(ep)(hmats[('s', name)], hmats[('d', name)],
                                    srcp, dstp)
            amat = _att_mat(p['att'])
            if aname is None:
                wm, ex = _edge_tc(ep, e, False)(ghs, ghd, amat, sel, eye)
            else:
                gd = (mvw[0] @ p['We']).reshape(1, HID)
                g0 = (mvb @ p['We']).reshape(1, HID)
                wm, ex = _edge_tc(ep, e, True)(ghs, ghd, eav, amat, sel, eye,
                                               g0, gd)
            nd = NDP[d]
            pn, pd_ = _scatter2(ep, nd)(wm, ex, dstp,
                                        jnp.zeros((CHUNK, HID), jnp.float32))
            pn_by_dst[d].append(pn)
            pd_by_dst[d].append(pd_)
            bsum[d] = bsum[d] + p['b']

        newx = {}
        for t in NN:
            k = len(pn_by_dst[t])
            g, b = lp['bn'][t]
            gs = (g / jnp.sqrt(1.0 + EPS)).reshape(1, HID)
            xp = jnp.pad(x[t], ((0, NDP[t] - NN[t]), (0, 0)))
            newx[t] = _combine_tc(NDP[t], k)(
                jnp.stack(pn_by_dst[t]), jnp.stack(pd_by_dst[t]), xp,
                bsum[t].reshape(1, HID), gs, b.reshape(1, HID), selp)[:NN[t]]
        x = newx

    w1, b1, w2, b2, w3, b3 = params['head']
    w3p = jnp.zeros((32, 8), jnp.float32).at[:, :1].set(w3)
    b3p = jnp.zeros((1, 8), jnp.float32).at[:, :1].set(b3.reshape(1, 1))
    vs = []
    attrs = []
    for s, name, d, aname in MV:
        _, _, _, e, ep, srcp, dstp, _ = einfo[name]
        g0, g1 = _gather2(ep)(x[s], x[d], srcp, dstp)
        v8 = _head_tc(ep)(g0, g1, w1, b1.reshape(1, 64), w2, b2.reshape(1, 32),
                          w3p, b3p)
        vs.append(v8[:e, 0])
        attrs.append(eas[aname])
    v = jnp.concatenate(vs, axis=0)
    a = jnp.concatenate(attrs, axis=0)
    masked = jnp.where(a[:, 0] == 1.0, v, -jnp.inf)
    av = jnp.full((NUM_ACTIONS,), -jnp.inf,
                  jnp.float32).at[move_to_action_indices].set(masked)
    return av[None, :], jnp.max(av).reshape(1, 1)


# speedup vs baseline: 18.5711x; 1.0645x over previous
import functools

import numpy as np
import jax
import jax.numpy as jnp
from jax import lax
from jax.experimental import pallas as pl
from jax.experimental.pallas import tpu as pltpu
from jax.experimental.pallas import tpu_sc as plsc

HID = 128
HEADS = 4
HD = HID // HEADS
NN = {'ip': 10000, 'op': 2000, 'de': 10000}
NB = [('ip', 'ei_ip_nb_ip', 'ip'), ('ip', 'ei_ip_nb_de', 'de'),
      ('de', 'ei_de_nb_ip', 'ip'), ('de', 'ei_de_nb_de', 'de'),
      ('ip', 'ei_ip_rnb_ip', 'ip'), ('de', 'ei_de_rnb_ip', 'ip'),
      ('ip', 'ei_ip_rnb_de', 'de'), ('de', 'ei_de_rnb_de', 'de')]
MV = [('ip', 'ei_ip_mv_de', 'de', 'ea_ip_mv_de'),
      ('op', 'ei_op_mv_de', 'de', 'ea_op_mv_de'),
      ('de', 'ei_de_rmv_ip', 'ip', 'ea_de_rmv_ip'),
      ('de', 'ei_de_rmv_op', 'op', 'ea_de_rmv_op')]
EPS = 1e-5
NUM_ACTIONS = 120000

ALL_ETYPES = [(s, name, d, None) for s, name, d in NB] + \
             [(s, name, d, aname) for s, name, d, aname in MV]

NC = 2
NS = 16
NW = NC * NS
CHUNK = 128
UNIT = NW * CHUNK

BM = 512
BN = 512
NDP = {'ip': 10240, 'op': 2048, 'de': 10240}

_SEL = np.zeros((16, HID), np.float32)
for _h in range(HEADS):
    _SEL[_h, _h * HD:(_h + 1) * HD] = 1.0

_EYE = np.zeros((16, HID), np.float32)
for _h in range(16):
    _EYE[_h, _h] = 1.0
_SELP = np.zeros((HID, HID), np.float32)
_SELP[:16, :] = _SEL


def _mesh():
    return plsc.VectorSubcoreMesh(core_axis_name="c", subcore_axis_name="s",
                                  num_cores=NC, num_subcores=NS)


def _pad_edges(n):
    return ((n + UNIT - 1) // UNIT) * UNIT


def _gather_pipe(hs_hbm, hd_hbm, si_hbm, di_hbm, ghs_hbm, ghd_hbm,
                 ia, ib, ra, rb, sga, sgb, swa, swb, base, nch):
    pend_g = [None, None]
    pend_w = [None, None]
    offs = [None, None]
    for i in range(nch):
        b = i & 1
        if pend_w[b] is not None:
            pend_w[b][0].wait()
            pend_w[b][1].wait()
            pend_w[b] = None
        off = base + i * CHUNK
        pltpu.sync_copy(si_hbm.at[pl.ds(off, CHUNK)], ia[b])
        pltpu.sync_copy(di_hbm.at[pl.ds(off, CHUNK)], ib[b])
        pend_g[b] = (pltpu.async_copy(hs_hbm.at[ia[b]], ra[b], sga[b]),
                     pltpu.async_copy(hd_hbm.at[ib[b]], rb[b], sgb[b]))
        offs[b] = off
        pb = (i - 1) & 1
        if i >= 1 and pend_g[pb] is not None:
            pend_g[pb][0].wait()
            pend_g[pb][1].wait()
            pend_g[pb] = None
            po = offs[pb]
            pend_w[pb] = (
                pltpu.async_copy(ra[pb], ghs_hbm.at[pl.ds(po, CHUNK)], swa[pb]),
                pltpu.async_copy(rb[pb], ghd_hbm.at[pl.ds(po, CHUNK)], swb[pb]))
    lb = (nch - 1) & 1
    pend_g[lb][0].wait()
    pend_g[lb][1].wait()
    pltpu.sync_copy(ra[lb], ghs_hbm.at[pl.ds(offs[lb], CHUNK)])
    pltpu.sync_copy(rb[lb], ghd_hbm.at[pl.ds(offs[lb], CHUNK)])
    for b in range(2):
        if pend_w[b] is not None:
            pend_w[b][0].wait()
            pend_w[b][1].wait()


@functools.lru_cache(None)
def _gatherN(eps):
    k = len(eps)

    def body(*refs):
        ins = refs[:4 * k]
        outs = refs[4 * k:6 * k]
        scr = refs[6 * k:]
        ia = scr[0:2]
        ib = scr[2:4]
        ra = scr[4:6]
        rb = scr[6:8]
        sga = scr[8:10]
        sgb = scr[10:12]
        swa = scr[12:14]
        swb = scr[14:16]
        wid = lax.axis_index("s") * NC + lax.axis_index("c")
        for t in range(k):
            bw = eps[t] // NW
            hs, hd, si, di = ins[4 * t:4 * t + 4]
            ghs, ghd = outs[2 * t:2 * t + 2]
            _gather_pipe(hs, hd, si, di, ghs, ghd, ia, ib, ra, rb,
                         sga, sgb, swa, swb, wid * bw, bw // CHUNK)

    out_type = []
    for ep in eps:
        out_type += [jax.ShapeDtypeStruct((ep, HID), jnp.float32),
                     jax.ShapeDtypeStruct((ep, HID), jnp.float32)]
    return pl.kernel(
        body,
        out_type=out_type,
        mesh=_mesh(),
        scratch_types=[pltpu.VMEM((CHUNK,), jnp.int32)] * 4 +
                      [pltpu.VMEM((CHUNK, HID), jnp.float32)] * 4 +
                      [pltpu.SemaphoreType.DMA] * 8,
    )


@functools.lru_cache(None)
def _scatter2(ep, nd):
    bw = ep // NW
    nch = bw // CHUNK
    nz = nd // NS
    nzc = nz // CHUNK

    def body(wm_hbm, ex_hbm, di_hbm, z_hbm, pn_hbm, pd_hbm,
             ix0, ix1, wv0, wv1, zv, tn, sa0, sa1):
        c = lax.axis_index("c")
        s = lax.axis_index("s")
        wid = s * NC + c
        base = wid * bw
        ix = (ix0, ix1)
        wv = (wv0, wv1)
        sa = (sa0, sa1)
        pltpu.sync_copy(z_hbm, zv)

        for phase in range(2):
            src_hbm = wm_hbm if phase == 0 else ex_hbm
            out_hbm = pn_hbm if phase == 0 else pd_hbm

            @pl.loop(0, nzc)
            def _(j):
                pltpu.sync_copy(zv, tn.at[pl.ds(s * nz + j * CHUNK, CHUNK)])

            plsc.subcore_barrier()

            @pl.loop(0, nch)
            def _(i):
                off = base + i * CHUNK
                pltpu.sync_copy(di_hbm.at[pl.ds(off, CHUNK)], ix0)
                pltpu.sync_copy(src_hbm.at[pl.ds(off, CHUNK)], wv0)
                pltpu.sync_copy(wv0, tn.at[ix0], add=True)

            plsc.subcore_barrier()

            @pl.loop(0, nzc)
            def _(j):
                o = s * nz + j * CHUNK
                pltpu.sync_copy(tn.at[pl.ds(o, CHUNK)], wv0)
                pltpu.sync_copy(wv0, out_hbm.at[c, pl.ds(o, CHUNK)])

            plsc.subcore_barrier()

    return pl.kernel(
        body,
        out_type=[jax.ShapeDtypeStruct((NC, nd, HID), jnp.float32),
                  jax.ShapeDtypeStruct((NC, nd, HID), jnp.float32)],
        mesh=_mesh(),
        scratch_types=[pltpu.VMEM((CHUNK,), jnp.int32),
                       pltpu.VMEM((CHUNK,), jnp.int32),
                       pltpu.VMEM((CHUNK, HID), jnp.float32),
                       pltpu.VMEM((CHUNK, HID), jnp.float32),
                       pltpu.VMEM((CHUNK, HID), jnp.float32),
                       pltpu.VMEM_SHARED((nd, HID), jnp.float32),
                       pltpu.SemaphoreType.DMA,
                       pltpu.SemaphoreType.DMA],
    )


@functools.lru_cache(None)
def _linear_tc(m, k, n, act, bm):
    nblk = m // bm

    def body(x_ref, w_ref, b_ref, o_ref):
        r = jnp.dot(x_ref[...], w_ref[...], preferred_element_type=jnp.float32)
        r = r + b_ref[...]
        if act == 'relu':
            r = jnp.maximum(r, 0.0)
        o_ref[...] = r

    return pl.pallas_call(
        body,
        grid=(nblk,),
        in_specs=[pl.BlockSpec((bm, k), lambda i: (i, 0)),
                  pl.BlockSpec((k, n), lambda i: (0, 0)),
                  pl.BlockSpec((1, n), lambda i: (0, 0))],
        out_specs=pl.BlockSpec((bm, n), lambda i: (i, 0)),
        out_shape=jax.ShapeDtypeStruct((m, n), jnp.float32),
    )


def _linear(x, w, b=None, act=None, bm=BM):
    m, k = x.shape
    n = w.shape[1]
    pad = (-m) % bm
    if pad:
        x = jnp.pad(x, ((0, pad), (0, 0)))
    if b is None:
        b = jnp.zeros((n,), jnp.float32)
    out = _linear_tc(x.shape[0], k, n, act, bm)(x, w, b.reshape(1, n))
    return out[:m] if pad else out


@functools.lru_cache(None)
def _edge_tc(ep, e_real, mv):
    nblk = ep // BM

    def body(*refs):
        if mv:
            ghs, ghd, ea, a_r, sel_r, eye_r, g0_r, gd_r, wm_o, ex_o = refs
        else:
            ghs, ghd, a_r, sel_r, eye_r, wm_o, ex_o = refs
        m = ghs[...] + ghd[...]
        if mv:
            m = m + ea[...] * gd_r[...] + g0_r[...]
        lr = jnp.where(m >= 0.0, m, 0.2 * m)
        ex16 = jnp.exp(jnp.dot(lr, a_r[...], preferred_element_type=jnp.float32))
        row = pl.program_id(0) * BM + lax.broadcasted_iota(jnp.int32, (BM, 1), 0)
        ex16 = jnp.where(row < e_real, ex16, 0.0)
        ex_o[...] = jnp.dot(ex16, eye_r[...], preferred_element_type=jnp.float32)
        wm_o[...] = ghs[...] * jnp.dot(ex16, sel_r[...],
                                       preferred_element_type=jnp.float32)

    ins = [pl.BlockSpec((BM, HID), lambda i: (i, 0)),
           pl.BlockSpec((BM, HID), lambda i: (i, 0))]
    if mv:
        ins.append(pl.BlockSpec((BM, 1), lambda i: (i, 0)))
    ins += [pl.BlockSpec((HID, 16), lambda i: (0, 0)),
            pl.BlockSpec((16, HID), lambda i: (0, 0)),
            pl.BlockSpec((16, HID), lambda i: (0, 0))]
    if mv:
        ins += [pl.BlockSpec((1, HID), lambda i: (0, 0)),
                pl.BlockSpec((1, HID), lambda i: (0, 0))]
    return pl.pallas_call(
        body,
        grid=(nblk,),
        in_specs=ins,
        out_specs=[pl.BlockSpec((BM, HID), lambda i: (i, 0)),
                   pl.BlockSpec((BM, HID), lambda i: (i, 0))],
        out_shape=[jax.ShapeDtypeStruct((ep, HID), jnp.float32),
                   jax.ShapeDtypeStruct((ep, HID), jnp.float32)],
    )


@functools.lru_cache(None)
def _combine_tc(n, k):
    nblk = n // BN

    def body(pn, pd_, x, bsum, gs, bb, sel, o):
        acc = None
        for e in range(k):
            num = pn[e, 0] + pn[e, 1]
            den = pd_[e, 0] + pd_[e, 1]
            denb = jnp.dot(den, sel[...],
                           preferred_element_type=jnp.float32) + 1e-16
            t = num / denb
            acc = t if acc is None else acc + t
        h = jnp.maximum(acc + bsum[...], 0.0) + x[...]
        o[...] = h * gs[...] + bb[...]

    return pl.pallas_call(
        body,
        grid=(nblk,),
        in_specs=[pl.BlockSpec((k, NC, BN, HID), lambda i: (0, 0, i, 0)),
                  pl.BlockSpec((k, NC, BN, HID), lambda i: (0, 0, i, 0)),
                  pl.BlockSpec((BN, HID), lambda i: (i, 0)),
                  pl.BlockSpec((1, HID), lambda i: (0, 0)),
                  pl.BlockSpec((1, HID), lambda i: (0, 0)),
                  pl.BlockSpec((1, HID), lambda i: (0, 0)),
                  pl.BlockSpec((HID, HID), lambda i: (0, 0))],
        out_specs=pl.BlockSpec((BN, HID), lambda i: (i, 0)),
        out_shape=jax.ShapeDtypeStruct((n, HID), jnp.float32),
    )


@functools.lru_cache(None)
def _head_tc(ep):
    nblk = ep // BM

    def body(g0, g1, w1, b1, w2, b2, w3, b3, o):
        f = (g0[...] + g1[...]) * 0.5
        h1 = jnp.maximum(jnp.dot(f, w1[...],
                                 preferred_element_type=jnp.float32) + b1[...], 0.0)
        h2 = jnp.maximum(jnp.dot(h1, w2[...],
                                 preferred_element_type=jnp.float32) + b2[...], 0.0)
        o[...] = jnp.tanh(jnp.dot(h2, w3[...],
                                  preferred_element_type=jnp.float32) + b3[...])

    return pl.pallas_call(
        body,
        grid=(nblk,),
        in_specs=[pl.BlockSpec((BM, HID), lambda i: (i, 0)),
                  pl.BlockSpec((BM, HID), lambda i: (i, 0)),
                  pl.BlockSpec((HID, 64), lambda i: (0, 0)),
                  pl.BlockSpec((1, 64), lambda i: (0, 0)),
                  pl.BlockSpec((64, 32), lambda i: (0, 0)),
                  pl.BlockSpec((1, 32), lambda i: (0, 0)),
                  pl.BlockSpec((32, 8), lambda i: (0, 0)),
                  pl.BlockSpec((1, 8), lambda i: (0, 0))],
        out_specs=pl.BlockSpec((BM, 8), lambda i: (i, 0)),
        out_shape=jax.ShapeDtypeStruct((ep, 8), jnp.float32),
    )


def _att_mat(att):
    cols = []
    for h in range(HEADS):
        cols.append(jnp.zeros((HD, 16), jnp.float32).at[:, h].set(att[h]))
    return jnp.concatenate(cols, axis=0)


def kernel(x_in_play, x_out_of_play, x_destination,
           ei_ip_nb_ip, ei_ip_nb_de, ei_de_nb_ip, ei_de_nb_de,
           ei_ip_rnb_ip, ei_de_rnb_ip, ei_ip_rnb_de, ei_de_rnb_de,
           ei_ip_mv_de, ei_op_mv_de, ei_de_rmv_ip, ei_de_rmv_op,
           ea_ip_mv_de, ea_op_mv_de, ea_de_rmv_ip, ea_de_rmv_op,
           move_to_action_indices, params):
    eis = {'ei_ip_nb_ip': ei_ip_nb_ip, 'ei_ip_nb_de': ei_ip_nb_de,
           'ei_de_nb_ip': ei_de_nb_ip, 'ei_de_nb_de': ei_de_nb_de,
           'ei_ip_rnb_ip': ei_ip_rnb_ip, 'ei_de_rnb_ip': ei_de_rnb_ip,
           'ei_ip_rnb_de': ei_ip_rnb_de, 'ei_de_rnb_de': ei_de_rnb_de,
           'ei_ip_mv_de': ei_ip_mv_de, 'ei_op_mv_de': ei_op_mv_de,
           'ei_de_rmv_ip': ei_de_rmv_ip, 'ei_de_rmv_op': ei_de_rmv_op}
    eas = {'ea_ip_mv_de': ea_ip_mv_de, 'ea_op_mv_de': ea_op_mv_de,
           'ea_de_rmv_ip': ea_de_rmv_ip, 'ea_de_rmv_op': ea_de_rmv_op}
    sel = jnp.asarray(_SEL)
    eye = jnp.asarray(_EYE)
    selp = jnp.asarray(_SELP)

    einfo = {}
    for s, name, d, aname in ALL_ETYPES:
        e = eis[name].shape[1]
        ep = _pad_edges(e)
        src = jnp.pad(eis[name][0].astype(jnp.int32), (0, ep - e))
        dst = jnp.pad(eis[name][1].astype(jnp.int32), (0, ep - e))
        eav = None
        if aname is not None:
            eav = jnp.pad(eas[aname].astype(jnp.float32), ((0, ep - e), (0, 0)))
        einfo[name] = (s, d, aname, e, ep, src, dst, eav)

    x = {'ip': _linear(x_in_play, *params['emb']['ip'], act='relu'),
         'op': _linear(x_out_of_play, *params['emb']['op'], act='relu'),
         'de': _linear(x_destination, *params['emb']['de'], act='relu')}

    mvw, mvb = params['mv_emb']

    for lp in params['layers']:
        conv = lp['conv']
        plan = {t: [] for t in NN}
        for s, name, d, aname in ALL_ETYPES:
            plan[s].append(('s', name, conv[name]['Ws']))
            plan[d].append(('d', name, conv[name]['Wd']))
        hmats = {}
        for t, items in plan.items():
            wcat = jnp.concatenate([w for _, _, w in items], axis=1)
            hcat = _linear(x[t], wcat)
            for j, (kind, name, _) in enumerate(items):
                hmats[(kind, name)] = hcat[:, j * HID:(j + 1) * HID]

        pn_by_dst = {t: [] for t in NN}
        pd_by_dst = {t: [] for t in NN}
        bsum = {t: jnp.zeros((HID,), jnp.float32) for t in NN}
        gin = []
        epl = []
        for s, name, d, aname in ALL_ETYPES:
            _, _, _, e, ep, srcp, dstp, eav = einfo[name]
            gin += [hmats[('s', name)], hmats[('d', name)], srcp, dstp]
            epl.append(ep)
        gout = _gatherN(tuple(epl))(*gin)
        for ti, (s, name, d, aname) in enumerate(ALL_ETYPES):
            _, _, _, e, ep, srcp, dstp, eav = einfo[name]
            p = conv[name]
            ghs, ghd = gout[2 * ti], gout[2 * ti + 1]
            amat = _att_mat(p['att'])
            if aname is None:
                wm, ex = _edge_tc(ep, e, False)(ghs, ghd, amat, sel, eye)
            else:
                gd = (mvw[0] @ p['We']).reshape(1, HID)
                g0 = (mvb @ p['We']).reshape(1, HID)
                wm, ex = _edge_tc(ep, e, True)(ghs, ghd, eav, amat, sel, eye,
                                               g0, gd)
            nd = NDP[d]
            pn, pd_ = _scatter2(ep, nd)(wm, ex, dstp,
                                        jnp.zeros((CHUNK, HID), jnp.float32))
            pn_by_dst[d].append(pn)
            pd_by_dst[d].append(pd_)
            bsum[d] = bsum[d] + p['b']

        newx = {}
        for t in NN:
            k = len(pn_by_dst[t])
            g, b = lp['bn'][t]
            gs = (g / jnp.sqrt(1.0 + EPS)).reshape(1, HID)
            xp = jnp.pad(x[t], ((0, NDP[t] - NN[t]), (0, 0)))
            newx[t] = _combine_tc(NDP[t], k)(
                jnp.stack(pn_by_dst[t]), jnp.stack(pd_by_dst[t]), xp,
                bsum[t].reshape(1, HID), gs, b.reshape(1, HID), selp)[:NN[t]]
        x = newx

    w1, b1, w2, b2, w3, b3 = params['head']
    w3p = jnp.zeros((32, 8), jnp.float32).at[:, :1].set(w3)
    b3p = jnp.zeros((1, 8), jnp.float32).at[:, :1].set(b3.reshape(1, 1))
    vs = []
    attrs = []
    gin = []
    epl = []
    for s, name, d, aname in MV:
        _, _, _, e, ep, srcp, dstp, _ = einfo[name]
        gin += [x[s], x[d], srcp, dstp]
        epl.append(ep)
    gout = _gatherN(tuple(epl))(*gin)
    for ti, (s, name, d, aname) in enumerate(MV):
        _, _, _, e, ep, srcp, dstp, _ = einfo[name]
        v8 = _head_tc(ep)(gout[2 * ti], gout[2 * ti + 1], w1,
                          b1.reshape(1, 64), w2, b2.reshape(1, 32), w3p, b3p)
        vs.append(v8[:e, 0])
        attrs.append(eas[aname])
    v = jnp.concatenate(vs, axis=0)
    a = jnp.concatenate(attrs, axis=0)
    masked = jnp.where(a[:, 0] == 1.0, v, -jnp.inf)
    av = jnp.full((NUM_ACTIONS,), -jnp.inf,
                  jnp.float32).at[move_to_action_indices].set(masked)
    return av[None, :], jnp.max(av).reshape(1, 1)
